# baseline (speedup 1.0000x reference)
import functools

import jax
import jax.numpy as jnp
from jax import lax
from jax.experimental import pallas as pl
from jax.experimental.pallas import tpu as pltpu
from jax.experimental.pallas import tpu_sc as plsc

_EMB = 32
_G = 128
_GPC = 10
_CH = _G * _GPC


def _make_gather(n_rows: int):
    info = plsc.get_sparse_core_info()
    nc, ns = info.num_cores, info.num_subcores
    nw = nc * ns
    b_per_w = n_rows // nw
    assert n_rows % (nw * _CH) == 0
    n_chunks = b_per_w // _CH
    groups_per_w = b_per_w // _G

    mesh = plsc.VectorSubcoreMesh(core_axis_name="c", subcore_axis_name="s")

    @functools.partial(
        pl.kernel,
        mesh=mesh,
        out_type=jax.ShapeDtypeStruct((n_rows, _EMB), jnp.float32),
        scratch_types=[
            pltpu.VMEM((b_per_w,), jnp.int32),
            pltpu.VMEM((_CH, _EMB), jnp.float32),
            pltpu.SemaphoreType.DMA,
        ],
        compiler_params=pltpu.CompilerParams(use_tc_tiling_on_sc=False),
    )
    def gather(idx_hbm, table_hbm, out_hbm, idx_v, rows_v, sem):
        wid = lax.axis_index("s") * nc + lax.axis_index("c")
        row_base = wid * b_per_w

        pltpu.sync_copy(idx_hbm.at[pl.ds(row_base, b_per_w)], idx_v)

        def chunk_body(c, _):
            copies = []
            for j in range(_GPC):
                cp = pltpu.make_async_copy(
                    table_hbm.at[idx_v.at[pl.ds((c * _GPC + j) * _G, _G)]],
                    rows_v.at[pl.ds(j * _G, _G)],
                    sem,
                )
                cp.start()
                copies.append(cp)
            for cp in copies:
                cp.wait()
            pltpu.sync_copy(
                rows_v, out_hbm.at[pl.ds(row_base + c * _CH, _CH)]
            )
            return ()

        lax.fori_loop(0, n_chunks, chunk_body, (), unroll=False)

    return gather


def kernel(indices, table):
    b, l = indices.shape
    n_rows = b * l
    idx_flat = indices.reshape(n_rows).astype(jnp.int32)
    out = _make_gather(n_rows)(idx_flat, table)
    return out.reshape(b, l, _EMB)


# speedup vs baseline: 1.2215x; 1.2215x over previous
import functools

import jax
import jax.numpy as jnp
from jax import lax
from jax.experimental import pallas as pl
from jax.experimental.pallas import tpu as pltpu
from jax.experimental.pallas import tpu_sc as plsc

_EMB = 32
_BC = 16


def _make_gather(batch: int, seq: int):
    info = plsc.get_sparse_core_info()
    nc, ns = info.num_cores, info.num_subcores
    nw = nc * ns
    rows_per_w = batch // nw
    assert batch % (nw * _BC) == 0
    n_chunks = rows_per_w // _BC

    mesh = plsc.VectorSubcoreMesh(core_axis_name="c", subcore_axis_name="s")

    @functools.partial(
        pl.kernel,
        mesh=mesh,
        out_type=jax.ShapeDtypeStruct((batch, seq, _EMB), jnp.float32),
        scratch_types=[
            pltpu.VMEM((rows_per_w, seq), jnp.int32),
            pltpu.VMEM((_BC, seq, _EMB), jnp.float32),
            pltpu.SemaphoreType.DMA,
        ],
        compiler_params=pltpu.CompilerParams(use_tc_tiling_on_sc=False),
    )
    def gather(idx_hbm, table_hbm, out_hbm, idx_v, rows_v, sem):
        wid = lax.axis_index("s") * nc + lax.axis_index("c")
        row_base = wid * rows_per_w

        pltpu.sync_copy(idx_hbm.at[pl.ds(row_base, rows_per_w)], idx_v)

        def chunk_body(c, _):
            copies = []
            for j in range(_BC):
                cp = pltpu.make_async_copy(
                    table_hbm.at[idx_v.at[c * _BC + j]],
                    rows_v.at[j],
                    sem,
                )
                cp.start()
                copies.append(cp)
            for cp in copies:
                cp.wait()
            pltpu.sync_copy(
                rows_v, out_hbm.at[pl.ds(row_base + c * _BC, _BC)]
            )
            return ()

        lax.fori_loop(0, n_chunks, chunk_body, (), unroll=False)

    return gather


def kernel(indices, table):
    b, l = indices.shape
    idx = indices.astype(jnp.int32)
    return _make_gather(b, l)(idx, table)


# speedup vs baseline: 1.2229x; 1.0012x over previous
import functools

import jax
import jax.numpy as jnp
from jax import lax
from jax.experimental import pallas as pl
from jax.experimental.pallas import tpu as pltpu
from jax.experimental.pallas import tpu_sc as plsc

_EMB = 32
_BC = 8


def _make_gather(batch: int, seq: int):
    info = plsc.get_sparse_core_info()
    nc, ns = info.num_cores, info.num_subcores
    nw = nc * ns
    rows_per_w = batch // nw
    assert batch % (nw * _BC * 2) == 0
    n_chunks = rows_per_w // _BC

    mesh = plsc.VectorSubcoreMesh(core_axis_name="c", subcore_axis_name="s")

    @functools.partial(
        pl.kernel,
        mesh=mesh,
        out_type=jax.ShapeDtypeStruct((batch, seq, _EMB), jnp.float32),
        scratch_types=[
            pltpu.VMEM((rows_per_w, seq), jnp.int32),
            pltpu.VMEM((_BC, seq, _EMB), jnp.float32),
            pltpu.VMEM((_BC, seq, _EMB), jnp.float32),
            pltpu.SemaphoreType.DMA,
            pltpu.SemaphoreType.DMA,
        ],
        compiler_params=pltpu.CompilerParams(use_tc_tiling_on_sc=False),
    )
    def gather(idx_hbm, table_hbm, out_hbm, idx_v, rv0, rv1, sem0, sem1):
        wid = lax.axis_index("s") * nc + lax.axis_index("c")
        row_base = wid * rows_per_w
        bufs = ((rv0, sem0), (rv1, sem1))

        pltpu.sync_copy(idx_hbm.at[pl.ds(row_base, rows_per_w)], idx_v)

        def fire(c, rv, sem):
            copies = []
            for j in range(_BC):
                cp = pltpu.make_async_copy(
                    table_hbm.at[idx_v.at[c * _BC + j]], rv.at[j], sem
                )
                cp.start()
                copies.append(cp)
            return copies

        def drain_store(c, copies, rv):
            for cp in copies:
                cp.wait()
            pltpu.sync_copy(rv, out_hbm.at[pl.ds(row_base + c * _BC, _BC)])

        def pair_body(p, _):
            c0 = p * 2
            cp0 = fire(c0, *bufs[0])
            cp1 = fire(c0 + 1, *bufs[1])
            drain_store(c0, cp0, bufs[0][0])
            drain_store(c0 + 1, cp1, bufs[1][0])
            return ()

        lax.fori_loop(0, n_chunks // 2, pair_body, (), unroll=False)

    return gather


def kernel(indices, table):
    b, l = indices.shape
    idx = indices.astype(jnp.int32)
    return _make_gather(b, l)(idx, table)
